# Initial kernel scaffold; baseline (speedup 1.0000x reference)
import jax
import jax.numpy as jnp
from jax.experimental import pallas as pl


def kernel(logits, targets):
    raise NotImplementedError("write your pallas kernel here")


# speedup vs baseline: 28.0833x; 28.0833x over previous
import functools

import jax
import jax.numpy as jnp
from jax.experimental import pallas as pl
from jax.experimental.pallas import tpu as pltpu

_THRESH = 0.7
_MIN_KEPT = 100000
_NCLS = 96
_BH = 32


def _loss_kernel(logits_ref, targets_ref, loss_ref, pres_ref):
    h = pl.program_id(1)
    x = logits_ref[...]
    t = targets_ref[...]
    m = jnp.max(x, axis=1, keepdims=True)
    e = jnp.exp(x - m)
    s = jnp.sum(e, axis=1)
    cls = jax.lax.broadcasted_iota(jnp.int32, x.shape, 1)
    oh = cls == t[:, None, :, :]
    et = jnp.sum(jnp.where(oh, e, 0.0), axis=1)
    p = et / s
    loss = -jnp.log(p + 1e-7)
    loss_ref[...] = loss
    ph = jnp.max(jnp.where(oh, 1.0, 0.0), axis=(2, 3))
    ph = jnp.pad(ph, ((0, 0), (0, 128 - _NCLS)))[:, None, :]

    @pl.when(h == 0)
    def _():
        pres_ref[...] = ph

    @pl.when(h != 0)
    def _():
        pres_ref[...] = jnp.maximum(pres_ref[...], ph)


def _select_kernel(loss_ref, pres_ref, out_ref, *, rank):
    lv = loss_ref[...]
    ib = jax.lax.bitcast_convert_type(lv, jnp.int32)
    ub = jax.lax.bitcast_convert_type(lv, jnp.uint32)
    key = jnp.where(ib >= 0, ub | jnp.uint32(0x80000000), ~ub)

    def body(i, p):
        shift = (jnp.uint32(31) - i.astype(jnp.uint32))
        cand = p + jax.lax.shift_left(jnp.uint32(1), shift)
        cnt = jnp.sum((key < cand).astype(jnp.int32))
        return jnp.where(cnt >= rank, p, cand)

    kstar = jax.lax.fori_loop(0, 32, body, jnp.uint32(0))
    msk = (key > kstar).astype(jnp.float32)
    out_ref[...] = msk
    pres = pres_ref[0, :, :]
    cols = jax.lax.iota(jnp.int32, 128)[None, :]
    first = jnp.maximum(msk[:, 0, :], jnp.where(cols < _NCLS, pres, 0.0))
    out_ref[:, 0, :] = first


def kernel(logits, targets):
    b, c, hh, ww = logits.shape
    n = hh * ww
    targets = targets.astype(jnp.int32)

    grid1 = (b, hh // _BH)
    loss, pres = pl.pallas_call(
        _loss_kernel,
        grid=grid1,
        in_specs=[
            pl.BlockSpec((1, c, _BH, ww), lambda i, j: (i, 0, j, 0)),
            pl.BlockSpec((1, _BH, ww), lambda i, j: (i, j, 0)),
        ],
        out_specs=[
            pl.BlockSpec((1, _BH, ww), lambda i, j: (i, j, 0)),
            pl.BlockSpec((1, 1, 128), lambda i, j: (i, 0, 0)),
        ],
        out_shape=[
            jax.ShapeDtypeStruct((b, hh, ww), jnp.float32),
            jax.ShapeDtypeStruct((b, 1, 128), jnp.float32),
        ],
        compiler_params=pltpu.CompilerParams(
            dimension_semantics=("parallel", "arbitrary"),
        ),
    )(logits, targets)

    rank = min(max(_MIN_KEPT, int(n * _THRESH)), n - 1) + 1
    rows = n // 128
    loss3 = loss.reshape(b, rows, 128)
    mask = pl.pallas_call(
        functools.partial(_select_kernel, rank=rank),
        grid=(b,),
        in_specs=[
            pl.BlockSpec((1, rows, 128), lambda i: (i, 0, 0)),
            pl.BlockSpec((1, 1, 128), lambda i: (i, 0, 0)),
        ],
        out_specs=pl.BlockSpec((1, rows, 128), lambda i: (i, 0, 0)),
        out_shape=jax.ShapeDtypeStruct((b, rows, 128), jnp.float32),
        compiler_params=pltpu.CompilerParams(
            dimension_semantics=("arbitrary",),
        ),
    )(loss3, pres)
    return mask.reshape(b, hh, ww)
